# baseline (device time: 9958 ns/iter reference)
import jax
import jax.numpy as jnp
from jax import lax
from jax.experimental import pallas as pl
from jax.experimental.pallas import tpu as pltpu


def kernel(x, dy, gamma):
    m, d = x.shape

    def body(x_ref, dy_ref, out_ref, partial_ref, recv_ref, send_sem, recv_sem):
        my_x = lax.axis_index("x")
        my_y = lax.axis_index("y")
        partner = (1 - my_x, my_y)

        xv = x_ref[:, :]
        dyv = dy_ref[:, :]
        mu = jnp.mean(xv, axis=1, keepdims=True)
        diff = xv - mu
        var = jnp.mean(diff * diff, axis=1, keepdims=True)
        rstd = lax.rsqrt(var + 1e-5)
        xhat = diff * rstd
        partial_ref[0, :] = jnp.sum(dyv * xhat, axis=0)
        partial_ref[1, :] = jnp.sum(dyv, axis=0)

        barrier_sem = pltpu.get_barrier_semaphore()
        pl.semaphore_signal(
            barrier_sem, inc=1,
            device_id=partner, device_id_type=pl.DeviceIdType.MESH,
        )
        pl.semaphore_wait(barrier_sem, 1)

        rdma = pltpu.make_async_remote_copy(
            src_ref=partial_ref,
            dst_ref=recv_ref,
            send_sem=send_sem,
            recv_sem=recv_sem,
            device_id=partner,
            device_id_type=pl.DeviceIdType.MESH,
        )
        rdma.start()
        rdma.wait()

        out_ref[:, :] = partial_ref[:, :] + recv_ref[:, :]

    return pl.pallas_call(
        body,
        out_shape=jax.ShapeDtypeStruct((2, d), jnp.float32),
        in_specs=[
            pl.BlockSpec(memory_space=pltpu.VMEM),
            pl.BlockSpec(memory_space=pltpu.VMEM),
        ],
        out_specs=pl.BlockSpec(memory_space=pltpu.VMEM),
        scratch_shapes=[
            pltpu.VMEM((2, d), jnp.float32),
            pltpu.VMEM((2, d), jnp.float32),
            pltpu.SemaphoreType.DMA,
            pltpu.SemaphoreType.DMA,
        ],
        compiler_params=pltpu.CompilerParams(collective_id=0),
    )(x, dy)


# device time: 8232 ns/iter; 1.2097x vs baseline; 1.2097x over previous
import jax
import jax.numpy as jnp
from jax import lax
from jax.experimental import pallas as pl
from jax.experimental.pallas import tpu as pltpu

_CHUNKS = 4


def kernel(x, dy, gamma):
    m, d = x.shape
    C = _CHUNKS
    BM = m // C
    x = pltpu.with_memory_space_constraint(x, pltpu.MemorySpace.HBM)
    dy = pltpu.with_memory_space_constraint(dy, pltpu.MemorySpace.HBM)

    def body(x_hbm, dy_hbm, out_ref, xv_ref, dyv_ref, partial_ref,
             recv_ref, xsems, ysems, send_sem, recv_sem):
        my_x = lax.axis_index("x")
        my_y = lax.axis_index("y")
        partner = (1 - my_x, my_y)

        cxs = [
            pltpu.make_async_copy(
                x_hbm.at[pl.ds(c * BM, BM), :],
                xv_ref.at[pl.ds(c * BM, BM), :],
                xsems.at[c],
            )
            for c in range(C)
        ]
        cys = [
            pltpu.make_async_copy(
                dy_hbm.at[pl.ds(c * BM, BM), :],
                dyv_ref.at[pl.ds(c * BM, BM), :],
                ysems.at[c],
            )
            for c in range(C)
        ]
        for c in range(C):
            cxs[c].start()
            cys[c].start()

        barrier_sem = pltpu.get_barrier_semaphore()
        pl.semaphore_signal(
            barrier_sem, inc=1,
            device_id=partner, device_id_type=pl.DeviceIdType.MESH,
        )

        dg = jnp.zeros((d,), jnp.float32)
        db = jnp.zeros((d,), jnp.float32)
        for c in range(C):
            cxs[c].wait()
            xv = xv_ref[pl.ds(c * BM, BM), :]
            mu = jnp.mean(xv, axis=1, keepdims=True)
            diff = xv - mu
            var = jnp.mean(diff * diff, axis=1, keepdims=True)
            rstd = lax.rsqrt(var + 1e-5)
            cys[c].wait()
            dyv = dyv_ref[pl.ds(c * BM, BM), :]
            dg = dg + jnp.sum(dyv * diff * rstd, axis=0)
            db = db + jnp.sum(dyv, axis=0)
        partial_ref[0, :] = dg
        partial_ref[1, :] = db

        pl.semaphore_wait(barrier_sem, 1)
        rdma = pltpu.make_async_remote_copy(
            src_ref=partial_ref, dst_ref=recv_ref,
            send_sem=send_sem, recv_sem=recv_sem,
            device_id=partner, device_id_type=pl.DeviceIdType.MESH,
        )
        rdma.start()
        rdma.wait()
        out_ref[:, :] = partial_ref[:, :] + recv_ref[:, :]

    return pl.pallas_call(
        body,
        out_shape=jax.ShapeDtypeStruct((2, d), jnp.float32),
        in_specs=[pl.BlockSpec(memory_space=pltpu.MemorySpace.HBM)] * 2,
        out_specs=pl.BlockSpec(memory_space=pltpu.VMEM),
        scratch_shapes=[
            pltpu.VMEM((m, d), jnp.float32),
            pltpu.VMEM((m, d), jnp.float32),
            pltpu.VMEM((2, d), jnp.float32),
            pltpu.VMEM((2, d), jnp.float32),
            pltpu.SemaphoreType.DMA((_CHUNKS,)),
            pltpu.SemaphoreType.DMA((_CHUNKS,)),
            pltpu.SemaphoreType.DMA,
            pltpu.SemaphoreType.DMA,
        ],
        compiler_params=pltpu.CompilerParams(collective_id=0),
    )(x, dy)
